# baseline (speedup 1.0000x reference)
import functools

import jax
import jax.numpy as jnp
from jax import lax
from jax.experimental import pallas as pl
from jax.experimental.pallas import tpu as pltpu
from jax.experimental.pallas import tpu_sc as plsc

N = 10000
NP = 10240
H = 128
NW = 32
RPT = NP // NW
CH = 512
NCH = 21
EFP = NW * NCH * CH
CAP = 11776
KB = 64
KB2 = 128
NEG = -3.0e38

_MESH = dict(core_axis_name="c", subcore_axis_name="s",
             num_cores=2, num_subcores=16)
_SC_PARAMS = pltpu.CompilerParams(needs_layout_passes=False)


def _wid():
    return lax.axis_index("s") * 2 + lax.axis_index("c")


def _to_bf16(a):
    np_ = a.shape[0]
    blk = 256

    def body(a_ref, o_ref):
        o_ref[...] = a_ref[...].astype(jnp.bfloat16)

    return pl.pallas_call(
        body,
        grid=(np_ // blk,),
        in_specs=[pl.BlockSpec((blk, np_), lambda i: (i, 0))],
        out_specs=pl.BlockSpec((blk, np_), lambda i: (i, 0)),
        out_shape=jax.ShapeDtypeStruct((np_, np_), jnp.bfloat16),
    )(a)


def _common_matmul(ab, bi=2048, bk=1024):
    np_ = ab.shape[0]

    def body(x_ref, y_ref, o_ref):
        @pl.when(pl.program_id(2) == 0)
        def _():
            o_ref[...] = jnp.zeros_like(o_ref)

        o_ref[...] += lax.dot_general(
            x_ref[...], y_ref[...], (((1,), (1,)), ((), ())),
            preferred_element_type=jnp.float32)

    return pl.pallas_call(
        body,
        grid=(np_ // bi, np_ // bi, np_ // bk),
        in_specs=[
            pl.BlockSpec((bi, bk), lambda i, j, k: (i, k)),
            pl.BlockSpec((bi, bk), lambda i, j, k: (j, k)),
        ],
        out_specs=pl.BlockSpec((bi, bi), lambda i, j, k: (i, j)),
        out_shape=jax.ShapeDtypeStruct((np_, np_), jnp.float32),
        compiler_params=pltpu.CompilerParams(
            dimension_semantics=("parallel", "parallel", "arbitrary")),
    )(ab, ab)


def _node_transform(h, W, Wa, bias=None, relu=False, bi=512):
    np_ = h.shape[0]

    def body(h_ref, w_ref, wa_ref, b_ref, xw_ref, uu_ref):
        hb = h_ref[...]
        if relu:
            hb = jnp.maximum(hb + b_ref[...], 0.0)
        xw = lax.dot_general(hb, w_ref[...], (((1,), (1,)), ((), ())),
                             preferred_element_type=jnp.float32)
        xw_ref[...] = xw
        war = wa_ref[...].reshape(2, H)
        uu_ref[...] = lax.dot_general(war, xw, (((1,), (1,)), ((), ())),
                                      preferred_element_type=jnp.float32)

    if bias is None:
        bias = jnp.zeros((1, H), jnp.float32)
    return pl.pallas_call(
        body,
        grid=(np_ // bi,),
        in_specs=[
            pl.BlockSpec((bi, H), lambda i: (i, 0)),
            pl.BlockSpec((H, H), lambda i: (0, 0)),
            pl.BlockSpec((1, 2 * H), lambda i: (0, 0)),
            pl.BlockSpec((1, H), lambda i: (0, 0)),
        ],
        out_specs=[
            pl.BlockSpec((bi, H), lambda i: (i, 0)),
            pl.BlockSpec((2, bi), lambda i: (0, i)),
        ],
        out_shape=[
            jax.ShapeDtypeStruct((np_, H), jnp.float32),
            jax.ShapeDtypeStruct((2, np_), jnp.float32),
        ],
    )(h, W, Wa.reshape(1, 2 * H), bias.reshape(1, H))


def _final(agg, b, Wout, bout, bi=256):
    np_ = agg.shape[0]
    C = Wout.shape[0]

    def body(a_ref, b_ref, w_ref, bo_ref, o_ref):
        hb = jnp.maximum(a_ref[...] + b_ref[...], 0.0)
        lg = lax.dot_general(hb, w_ref[...], (((1,), (1,)), ((), ())),
                             preferred_element_type=jnp.float32)
        lg = lg + bo_ref[...]
        m = jnp.max(lg, axis=-1, keepdims=True)
        ls = lg - m
        o_ref[...] = ls - jnp.log(jnp.sum(jnp.exp(ls), axis=-1, keepdims=True))

    return pl.pallas_call(
        body,
        grid=(np_ // bi,),
        in_specs=[
            pl.BlockSpec((bi, H), lambda i: (i, 0)),
            pl.BlockSpec((1, H), lambda i: (0, 0)),
            pl.BlockSpec((C, H), lambda i: (0, 0)),
            pl.BlockSpec((1, C), lambda i: (0, 0)),
        ],
        out_specs=pl.BlockSpec((bi, C), lambda i: (i, 0)),
        out_shape=jax.ShapeDtypeStruct((np_, C), jnp.float32),
    )(agg, b.reshape(1, H), Wout, bout.reshape(1, C))


def _scatter_adj(src, dst, a_ref):
    @functools.partial(
        pl.kernel,
        mesh=plsc.VectorSubcoreMesh(**_MESH),
        out_type=(),
        compiler_params=_SC_PARAMS,
        scratch_types=[
            pltpu.VMEM((CH,), jnp.int32),
            pltpu.VMEM((CH,), jnp.int32),
            pltpu.VMEM((CH // 128, 128), jnp.int32),
            pltpu.VMEM((CH // 128, 128), jnp.int32),
            pltpu.VMEM((128,), jnp.float32),
            pltpu.SemaphoreType.DMA,
        ],
    )
    def k(src_hbm, dst_hbm, a_hbm, sbuf, dbuf, ibuf, rbuf, ones, sem):
        base0 = _wid() * NCH * CH
        for i in range(8):
            ones[pl.ds(i * 16, 16)] = jnp.full((16,), 1.0, jnp.float32)

        def chunk(c, _):
            base = base0 + c * CH
            cs = pltpu.async_copy(src_hbm.at[pl.ds(base, CH)], sbuf, sem)
            cd = pltpu.async_copy(dst_hbm.at[pl.ds(base, CH)], dbuf, sem)
            cs.wait()
            cd.wait()
            for q in range(CH // 128):
                for v in range(8):
                    sl = pl.ds(q * 128 + v * 16, 16)
                    s = sbuf[sl]
                    d = dbuf[sl]
                    ibuf[q, pl.ds(v * 16, 16)] = s * NP + d
                    rbuf[q, pl.ds(v * 16, 16)] = d * NP + s
            copies = []
            for q in range(CH // 128):
                copies.append(
                    pltpu.async_copy(ones, a_hbm.at[ibuf.at[q]], sem))
                copies.append(
                    pltpu.async_copy(ones, a_hbm.at[rbuf.at[q]], sem))
            for cp in copies:
                cp.wait()
            return 0

        lax.fori_loop(0, NCH, chunk, 0)

    k(src, dst, a_ref)


def _edge_scalars(src, dst, common, uu, pars, with_sim):
    out_type = [
        jax.ShapeDtypeStruct((EFP,), jnp.float32),
        jax.ShapeDtypeStruct((2, NP), jnp.float32),
    ]
    scratch = [
        pltpu.VMEM((NP,), jnp.float32),
        pltpu.VMEM((NP,), jnp.float32),
        pltpu.VMEM((CH,), jnp.int32),
        pltpu.VMEM((CH,), jnp.int32),
        pltpu.VMEM((CH // 128, 128), jnp.int32),
        pltpu.VMEM((CH,), jnp.float32),
        pltpu.VMEM((1024,), jnp.float32),
        pltpu.VMEM((16,), jnp.float32),
        pltpu.VMEM_SHARED((NP,), jnp.float32),
        pltpu.SemaphoreType.DMA,
    ]
    if with_sim:
        out_type += [
            jax.ShapeDtypeStruct((EFP,), jnp.float32),
            jax.ShapeDtypeStruct((2, NP), jnp.float32),
        ]
        scratch += [
            pltpu.VMEM((NP,), jnp.float32),
            pltpu.VMEM((CH // 128, 128), jnp.int32),
            pltpu.VMEM((CH,), jnp.float32),
            pltpu.VMEM((CH // 128, 128), jnp.int32),
            pltpu.VMEM((CH,), jnp.float32),
            pltpu.VMEM((NP // 16 // 128, 128), jnp.int32),
            pltpu.VMEM((NP // 16 // 128, 128), jnp.float32),
            pltpu.VMEM_SHARED((NP,), jnp.float32),
            pltpu.VMEM_SHARED((NP,), jnp.float32),
        ]

    def body(*refs):
        if with_sim:
            (src_hbm, dst_hbm, uu_hbm, pars_hbm, common_hbm,
             e_hbm, esum_hbm, ssim_hbm, ssum_hbm,
             u1t, u2t, sbuf, dbuf, dbufw, ebuf, zbuf, parv, eacc, sem,
             degt, cidx, interb, sbufw, ssimb, didx, dvals, sacc,
             degs_sh) = refs
        else:
            (src_hbm, dst_hbm, uu_hbm, pars_hbm,
             e_hbm, esum_hbm,
             u1t, u2t, sbuf, dbuf, dbufw, ebuf, zbuf, parv, eacc,
             sem) = refs

        cid = lax.axis_index("c")
        sid = lax.axis_index("s")
        base0 = _wid() * NCH * CH
        nq = CH // 128

        pltpu.sync_copy(uu_hbm.at[0], u1t)
        pltpu.sync_copy(uu_hbm.at[1], u2t)
        pltpu.sync_copy(pars_hbm, parv)
        ba = parv[pl.ds(0, 16)][0]

        for i in range(64):
            zbuf[pl.ds(i * 16, 16)] = jnp.zeros((16,), jnp.float32)

        @pl.when(sid == 0)
        def _():
            for r in range(NP // 1024):
                pltpu.sync_copy(zbuf, eacc.at[pl.ds(r * 1024, 1024)])
                if with_sim:
                    pltpu.sync_copy(zbuf, sacc.at[pl.ds(r * 1024, 1024)])

        if with_sim:
            perq = NP // 16 // 128
            dlo = sid * (NP // 16)
            iota = lax.iota(jnp.int32, 16)
            for q in range(perq):
                for v in range(8):
                    row = dlo + q * 128 + v * 16 + iota
                    didx[q, pl.ds(v * 16, 16)] = row * (NP + 1)
            for q in range(perq):
                pltpu.async_copy(common_hbm.at[didx.at[q]],
                                 dvals.at[q], sem).wait()
            for q in range(perq):
                pltpu.sync_copy(dvals.at[q],
                                degs_sh.at[pl.ds(dlo + q * 128, 128)])

        plsc.subcore_barrier()
        if with_sim:
            pltpu.sync_copy(degs_sh, degt)

        def chunk(c, _):
            base = base0 + c * CH
            c1 = pltpu.async_copy(src_hbm.at[pl.ds(base, CH)], sbuf, sem)
            c2 = pltpu.async_copy(dst_hbm.at[pl.ds(base, CH)], dbuf, sem)
            c1.wait()
            c2.wait()
            if with_sim:
                def gidx(g, _):
                    q = g // 8
                    v = g % 8
                    sl = pl.ds(g * 16, 16)
                    vs = pl.ds(v * 16, 16)
                    s = sbuf[sl]
                    d = dbuf[sl]
                    cidx[q, vs] = s * NP + d
                    dbufw[q, vs] = d
                    sbufw[q, vs] = s
                    return 0

                lax.fori_loop(0, CH // 16, gidx, 0)
                for q in range(nq):
                    pltpu.async_copy(
                        common_hbm.at[cidx.at[q]],
                        interb.at[pl.ds(q * 128, 128)], sem).wait()
            else:
                def gidx2(g, _):
                    q = g // 8
                    v = g % 8
                    dbufw[q, pl.ds(v * 16, 16)] = dbuf[pl.ds(g * 16, 16)]
                    return 0

                lax.fori_loop(0, CH // 16, gidx2, 0)

            def gcompute(g, _):
                sl = pl.ds(g * 16, 16)
                s = sbuf[sl]
                d = dbuf[sl]
                u1v = plsc.load_gather(u1t, [s])
                u2v = plsc.load_gather(u2t, [d])
                lg = u1v + u2v + ba
                ebuf[sl] = jnp.exp(jnp.where(lg >= 0.0, lg, lg * 0.2))
                if with_sim:
                    inter = interb[sl]
                    degs = plsc.load_gather(degt, [s])
                    degd = plsc.load_gather(degt, [d])
                    union = degs + degd - inter
                    ssimb[sl] = jnp.exp(inter / union)
                return 0

            lax.fori_loop(0, CH // 16, gcompute, 0)
            wcp = [pltpu.async_copy(ebuf, e_hbm.at[pl.ds(base, CH)], sem)]
            if with_sim:
                wcp.append(pltpu.async_copy(
                    ssimb, ssim_hbm.at[pl.ds(base, CH)], sem))
            for q in range(nq):
                pltpu.sync_copy(ebuf.at[pl.ds(q * 128, 128)],
                                eacc.at[dbufw.at[q]], add=True)
                if with_sim:
                    pltpu.sync_copy(ssimb.at[pl.ds(q * 128, 128)],
                                    sacc.at[sbufw.at[q]], add=True)
            for cp in wcp:
                cp.wait()
            return 0

        lax.fori_loop(0, NCH, chunk, 0)
        plsc.subcore_barrier()

        @pl.when(sid == 0)
        def _():
            pltpu.sync_copy(eacc, esum_hbm.at[cid])
            if with_sim:
                pltpu.sync_copy(sacc, ssum_hbm.at[cid])

    kern = functools.partial(
        pl.kernel,
        mesh=plsc.VectorSubcoreMesh(**_MESH),
        out_type=out_type,
        scratch_types=scratch,
        compiler_params=_SC_PARAMS,
    )(body)
    if with_sim:
        return kern(src, dst, uu, pars, common)
    return kern(src, dst, uu, pars)


def _aggregate(src, dst, e, ssim, esum_p, ssum_p, xw, pars):
    @functools.partial(
        pl.kernel,
        mesh=plsc.VectorSubcoreMesh(**_MESH),
        out_type=jax.ShapeDtypeStruct((NP * H,), jnp.float32),
        compiler_params=_SC_PARAMS,
        scratch_types=[
            pltpu.VMEM((NP,), jnp.float32),
            pltpu.VMEM((NP,), jnp.float32),
            pltpu.VMEM((NP,), jnp.float32),
            pltpu.VMEM((RPT * H,), jnp.float32),
            pltpu.VMEM((CAP,), jnp.int32),
            pltpu.VMEM((CAP,), jnp.float32),
            pltpu.VMEM((CAP,), jnp.int32),
            pltpu.VMEM((KB2, H), jnp.float32),
            pltpu.VMEM((8, H), jnp.float32),
            pltpu.VMEM((CH,), jnp.int32),
            pltpu.VMEM((CH,), jnp.int32),
            pltpu.VMEM((CH,), jnp.float32),
            pltpu.VMEM((CH,), jnp.float32),
            pltpu.VMEM((16,), jnp.float32),
            pltpu.SemaphoreType.DMA,
            pltpu.SemaphoreType.DMA,
        ],
    )
    def k(src_hbm, dst_hbm, e_hbm, ssim_hbm, esum_hbm, ssum_hbm, xw_hbm,
          pars_hbm, agg_hbm,
          esumt, ssumt, tmpt, acc, csrc, ccoef, coff, rows0, rows1,
          sbuf, dbuf, ebuf, ssimb, parv, sem0, sem1):
        wid = _wid()
        lo = wid * RPT
        base0 = wid * NCH * CH

        pltpu.sync_copy(pars_hbm, parv)
        al = jnp.minimum(jnp.maximum(parv[pl.ds(0, 16)][0], 0.0001), 0.9999)
        al1 = 1.0 - al

        pltpu.sync_copy(esum_hbm.at[0], esumt)
        pltpu.sync_copy(esum_hbm.at[1], tmpt)

        def addt(i, _):
            sl = pl.ds(i * 16, 16)
            esumt[sl] = esumt[sl] + tmpt[sl]
            return 0

        lax.fori_loop(0, NP // 16, addt, 0)
        pltpu.sync_copy(ssum_hbm.at[0], ssumt)
        pltpu.sync_copy(ssum_hbm.at[1], tmpt)

        def addt2(i, _):
            sl = pl.ds(i * 16, 16)
            ssumt[sl] = ssumt[sl] + tmpt[sl]
            return 0

        lax.fori_loop(0, NP // 16, addt2, 0)

        def initacc(i, _):
            acc[pl.ds(i * 16, 16)] = jnp.full((16,), NEG, jnp.float32)
            return 0

        lax.fori_loop(0, RPT * H // 16, initacc, 0)

        def initsrc(i, _):
            csrc[pl.ds(i * 16, 16)] = jnp.zeros((16,), jnp.int32)
            return 0

        lax.fori_loop(0, CAP // 16, initsrc, 0)

        def chunk(c, cnt):
            base = base0 + c * CH
            cps = [pltpu.async_copy(src_hbm.at[pl.ds(base, CH)], sbuf, sem0),
                   pltpu.async_copy(dst_hbm.at[pl.ds(base, CH)], dbuf, sem0),
                   pltpu.async_copy(e_hbm.at[pl.ds(base, CH)], ebuf, sem0),
                   pltpu.async_copy(ssim_hbm.at[pl.ds(base, CH)], ssimb,
                                    sem0)]
            for cp in cps:
                cp.wait()
            def group(v, cnt):
                sl = pl.ds(v * 16, 16)
                s = sbuf[sl]
                d = dbuf[sl]
                m = (d >= lo) & (d < lo + RPT) & (d < N)
                es = plsc.load_gather(esumt, [d])
                ss = plsc.load_gather(ssumt, [d])
                coef = al * ebuf[sl] / es + al1 * ssimb[sl] / ss
                off = (d - lo) * H
                plsc.store_compressed(csrc.at[pl.ds(cnt, 16)], s, mask=m)
                plsc.store_compressed(ccoef.at[pl.ds(cnt, 16)], coef, mask=m)
                plsc.store_compressed(coff.at[pl.ds(cnt, 16)], off, mask=m)
                return jnp.minimum(
                    cnt + jnp.sum(m.astype(jnp.int32)), CAP - 16)

            return lax.fori_loop(0, CH // 16, group, cnt)

        cnt = lax.fori_loop(0, NCH, chunk, jnp.int32(0))

        def batch(b, _):
            pltpu.async_copy(
                xw_hbm.at[csrc.at[pl.ds(b * KB2, KB2)]], rows0, sem0).wait()
            nbv = jnp.maximum(jnp.minimum(cnt - b * KB2, KB2), 0)

            def slot(t, _):
                off = coff[pl.ds(b * KB2 + t, 16)][0]
                cf = ccoef[pl.ds(b * KB2 + t, 16)][0]
                for j in range(H // 16):
                    sl = pl.ds(off + j * 16, 16)
                    acc[sl] = jnp.maximum(acc[sl],
                                          rows0[t, pl.ds(j * 16, 16)] * cf)
                return 0

            lax.fori_loop(0, nbv, slot, 0)
            return 0

        lax.fori_loop(0, (cnt + KB2 - 1) // KB2, batch, 0)
        pltpu.sync_copy(acc, agg_hbm.at[pl.ds(lo * H, RPT * H)])

    return k(src, dst, e, ssim, esum_p, ssum_p, xw, pars)


def kernel(x, edge_index, W1, Wa1, ba1, alpha1, b1, W2, Wa2, ba2, alpha2, b2,
           Wout, bout):
    loops = jnp.arange(N, dtype=edge_index.dtype)
    src = jnp.concatenate([edge_index[0], loops])
    dst = jnp.concatenate([edge_index[1], loops])
    npad = EFP - src.shape[0]
    fill = jnp.full((npad,), N, edge_index.dtype)
    src = jnp.concatenate([src, fill])
    dst = jnp.concatenate([dst, fill])

    xp = jnp.pad(x, ((0, NP - N), (0, 0)))

    a_ref = jax.new_ref(jnp.zeros((NP * NP,), jnp.float32))
    _scatter_adj(src, dst, a_ref)
    A = a_ref[...].reshape(NP, NP)
    common = _common_matmul(_to_bf16(A)).reshape(-1)

    def pars16(*vals):
        v = jnp.concatenate([jnp.asarray(t, jnp.float32).reshape(-1)
                             for t in vals])
        return jnp.pad(v, (0, 16 - v.shape[0]))

    xw1, uu1 = _node_transform(xp, W1, Wa1)
    e1, esum1, ssim, ssum = _edge_scalars(src, dst, common, uu1,
                                          pars16(ba1), with_sim=True)
    agg1 = _aggregate(src, dst, e1, ssim, esum1, ssum, xw1, pars16(alpha1))

    xw2, uu2 = _node_transform(agg1.reshape(NP, H), W2, Wa2,
                               bias=b1, relu=True)
    e2, esum2 = _edge_scalars(src, dst, None, uu2, pars16(ba2),
                              with_sim=False)
    agg2 = _aggregate(src, dst, e2, ssim, esum2, ssum, xw2, pars16(alpha2))

    out = _final(agg2.reshape(NP, H), b2, Wout, bout)
    return out[:N]


# speedup vs baseline: 1.5647x; 1.5647x over previous
import functools

import jax
import jax.numpy as jnp
from jax import lax
from jax.experimental import pallas as pl
from jax.experimental.pallas import tpu as pltpu
from jax.experimental.pallas import tpu_sc as plsc

N = 10000
NP = 10240
H = 128
NW = 32
RPT = NP // NW
CH = 128
EFP = 331776
CAP = 12352
NEG = -3.0e38

_MESH = dict(core_axis_name="c", subcore_axis_name="s",
             num_cores=2, num_subcores=16)


def _wid():
    return lax.axis_index("s") * 2 + lax.axis_index("c")


def _to_bf16(a):
    np_ = a.shape[0]
    blk = 256

    def body(a_ref, o_ref):
        o_ref[...] = a_ref[...].astype(jnp.bfloat16)

    return pl.pallas_call(
        body,
        grid=(np_ // blk,),
        in_specs=[pl.BlockSpec((blk, np_), lambda i: (i, 0))],
        out_specs=pl.BlockSpec((blk, np_), lambda i: (i, 0)),
        out_shape=jax.ShapeDtypeStruct((np_, np_), jnp.bfloat16),
    )(a)


def _common_matmul(ab, bi=2048, bk=1024):
    np_ = ab.shape[0]

    def body(x_ref, y_ref, o_ref):
        @pl.when(pl.program_id(2) == 0)
        def _():
            o_ref[...] = jnp.zeros_like(o_ref)

        o_ref[...] += lax.dot_general(
            x_ref[...], y_ref[...], (((1,), (1,)), ((), ())),
            preferred_element_type=jnp.float32)

    return pl.pallas_call(
        body,
        grid=(np_ // bi, np_ // bi, np_ // bk),
        in_specs=[
            pl.BlockSpec((bi, bk), lambda i, j, k: (i, k)),
            pl.BlockSpec((bi, bk), lambda i, j, k: (j, k)),
        ],
        out_specs=pl.BlockSpec((bi, bi), lambda i, j, k: (i, j)),
        out_shape=jax.ShapeDtypeStruct((np_, np_), jnp.float32),
        compiler_params=pltpu.CompilerParams(
            dimension_semantics=("parallel", "parallel", "arbitrary")),
    )(ab, ab)


def _node_transform(h, W, Wa, bias=None, relu=False, bi=512):
    np_ = h.shape[0]

    def body(h_ref, w_ref, wa_ref, b_ref, xw_ref, uu_ref):
        hb = h_ref[...]
        if relu:
            hb = jnp.maximum(hb + b_ref[...], 0.0)
        xw = lax.dot_general(hb, w_ref[...], (((1,), (1,)), ((), ())),
                             preferred_element_type=jnp.float32)
        xw_ref[...] = xw
        war = wa_ref[...].reshape(2, H)
        uu_ref[...] = lax.dot_general(war, xw, (((1,), (1,)), ((), ())),
                                      preferred_element_type=jnp.float32)

    if bias is None:
        bias = jnp.zeros((1, H), jnp.float32)
    return pl.pallas_call(
        body,
        grid=(np_ // bi,),
        in_specs=[
            pl.BlockSpec((bi, H), lambda i: (i, 0)),
            pl.BlockSpec((H, H), lambda i: (0, 0)),
            pl.BlockSpec((1, 2 * H), lambda i: (0, 0)),
            pl.BlockSpec((1, H), lambda i: (0, 0)),
        ],
        out_specs=[
            pl.BlockSpec((bi, H), lambda i: (i, 0)),
            pl.BlockSpec((2, bi), lambda i: (0, i)),
        ],
        out_shape=[
            jax.ShapeDtypeStruct((np_, H), jnp.float32),
            jax.ShapeDtypeStruct((2, np_), jnp.float32),
        ],
    )(h, W, Wa.reshape(1, 2 * H), bias.reshape(1, H))


def _final(agg, b, Wout, bout, bi=256):
    np_ = agg.shape[0]
    C = Wout.shape[0]

    def body(a_ref, b_ref, w_ref, bo_ref, o_ref):
        hb = jnp.maximum(a_ref[...] + b_ref[...], 0.0)
        lg = lax.dot_general(hb, w_ref[...], (((1,), (1,)), ((), ())),
                             preferred_element_type=jnp.float32)
        lg = lg + bo_ref[...]
        m = jnp.max(lg, axis=-1, keepdims=True)
        ls = lg - m
        o_ref[...] = ls - jnp.log(jnp.sum(jnp.exp(ls), axis=-1, keepdims=True))

    return pl.pallas_call(
        body,
        grid=(np_ // bi,),
        in_specs=[
            pl.BlockSpec((bi, H), lambda i: (i, 0)),
            pl.BlockSpec((1, H), lambda i: (0, 0)),
            pl.BlockSpec((C, H), lambda i: (0, 0)),
            pl.BlockSpec((1, C), lambda i: (0, 0)),
        ],
        out_specs=pl.BlockSpec((bi, C), lambda i: (i, 0)),
        out_shape=jax.ShapeDtypeStruct((np_, C), jnp.float32),
    )(agg, b.reshape(1, H), Wout, bout.reshape(1, C))


def _scatter_adj(src, dst, a_ref):
    nch = EFP // (NW * CH)

    @functools.partial(
        pl.kernel,
        mesh=plsc.VectorSubcoreMesh(**_MESH),
        out_type=(),
        compiler_params=pltpu.CompilerParams(needs_layout_passes=False),
        scratch_types=[
            pltpu.VMEM((CH,), jnp.int32),
            pltpu.VMEM((CH,), jnp.int32),
            pltpu.VMEM((CH,), jnp.int32),
            pltpu.VMEM((CH,), jnp.int32),
            pltpu.VMEM((CH,), jnp.float32),
            pltpu.SemaphoreType.DMA,
        ],
    )
    def k(src_hbm, dst_hbm, a_hbm, sbuf, dbuf, ibuf, rbuf, ones, sem):
        base0 = _wid() * nch * CH
        for i in range(CH // 16):
            ones[pl.ds(i * 16, 16)] = jnp.full((16,), 1.0, jnp.float32)

        def chunk(c, _):
            base = base0 + c * CH
            pltpu.sync_copy(src_hbm.at[pl.ds(base, CH)], sbuf)
            pltpu.sync_copy(dst_hbm.at[pl.ds(base, CH)], dbuf)
            for v in range(CH // 16):
                s = sbuf[pl.ds(v * 16, 16)]
                d = dbuf[pl.ds(v * 16, 16)]
                ibuf[pl.ds(v * 16, 16)] = s * NP + d
                rbuf[pl.ds(v * 16, 16)] = d * NP + s
            pltpu.async_copy(ones, a_hbm.at[ibuf], sem).wait()
            pltpu.async_copy(ones, a_hbm.at[rbuf], sem).wait()
            return 0

        lax.fori_loop(0, nch, chunk, 0)

    k(src, dst, a_ref)


def _edge_scalars(src, dst, common, uu, pars, with_sim):
    nch = EFP // (NW * CH)

    out_type = [
        jax.ShapeDtypeStruct((EFP,), jnp.float32),
        jax.ShapeDtypeStruct((2, NP), jnp.float32),
    ]
    scratch = [
        pltpu.VMEM((NP,), jnp.float32),
        pltpu.VMEM((NP,), jnp.float32),
        pltpu.VMEM((CH,), jnp.int32),
        pltpu.VMEM((CH,), jnp.int32),
        pltpu.VMEM((CH,), jnp.float32),
        pltpu.VMEM((1024,), jnp.float32),
        pltpu.VMEM((16,), jnp.float32),
        pltpu.VMEM_SHARED((NP,), jnp.float32),
        pltpu.SemaphoreType.DMA,
    ]
    if with_sim:
        out_type += [
            jax.ShapeDtypeStruct((EFP,), jnp.float32),
            jax.ShapeDtypeStruct((2, NP), jnp.float32),
        ]
        scratch += [
            pltpu.VMEM((CH,), jnp.int32),
            pltpu.VMEM((CH,), jnp.int32),
            pltpu.VMEM((CH,), jnp.int32),
            pltpu.VMEM((CH,), jnp.float32),
            pltpu.VMEM((CH,), jnp.float32),
            pltpu.VMEM((CH,), jnp.float32),
            pltpu.VMEM((CH,), jnp.float32),
            pltpu.VMEM_SHARED((NP,), jnp.float32),
        ]

    def body(*refs):
        if with_sim:
            (src_hbm, dst_hbm, uu_hbm, pars_hbm, common_hbm,
             e_hbm, esum_hbm, ssim_hbm, ssum_hbm,
             u1t, u2t, sbuf, dbuf, ebuf, zbuf, parv, eacc, sem,
             cidx, sidx, didx, interb, degsb, degdb, ssimb, sacc) = refs
        else:
            (src_hbm, dst_hbm, uu_hbm, pars_hbm,
             e_hbm, esum_hbm,
             u1t, u2t, sbuf, dbuf, ebuf, zbuf, parv, eacc, sem) = refs

        cid = lax.axis_index("c")
        sid = lax.axis_index("s")
        base0 = _wid() * nch * CH

        pltpu.sync_copy(uu_hbm.at[0], u1t)
        pltpu.sync_copy(uu_hbm.at[1], u2t)
        pltpu.sync_copy(pars_hbm, parv)
        ba = parv[pl.ds(0, 16)][0]

        for i in range(64):
            zbuf[pl.ds(i * 16, 16)] = jnp.zeros((16,), jnp.float32)

        @pl.when(sid == 0)
        def _():
            for r in range(NP // 1024):
                pltpu.sync_copy(zbuf, eacc.at[pl.ds(r * 1024, 1024)])
                if with_sim:
                    pltpu.sync_copy(zbuf, sacc.at[pl.ds(r * 1024, 1024)])

        plsc.subcore_barrier()

        def chunk(c, _):
            base = base0 + c * CH
            pltpu.sync_copy(src_hbm.at[pl.ds(base, CH)], sbuf)
            pltpu.sync_copy(dst_hbm.at[pl.ds(base, CH)], dbuf)
            if with_sim:
                for v in range(CH // 16):
                    s = sbuf[pl.ds(v * 16, 16)]
                    d = dbuf[pl.ds(v * 16, 16)]
                    cidx[pl.ds(v * 16, 16)] = s * NP + d
                    sidx[pl.ds(v * 16, 16)] = s * (NP + 1)
                    didx[pl.ds(v * 16, 16)] = d * (NP + 1)
                pltpu.async_copy(common_hbm.at[cidx], interb, sem).wait()
                pltpu.async_copy(common_hbm.at[sidx], degsb, sem).wait()
                pltpu.async_copy(common_hbm.at[didx], degdb, sem).wait()
            for v in range(CH // 16):
                sl = pl.ds(v * 16, 16)
                s = sbuf[sl]
                d = dbuf[sl]
                u1v = plsc.load_gather(u1t, [s])
                u2v = plsc.load_gather(u2t, [d])
                lg = u1v + u2v + ba
                ebuf[sl] = jnp.exp(jnp.where(lg >= 0.0, lg, lg * 0.2))
                if with_sim:
                    inter = interb[sl]
                    union = degsb[sl] + degdb[sl] - inter
                    ssimb[sl] = jnp.exp(inter / union)
            pltpu.sync_copy(ebuf, e_hbm.at[pl.ds(base, CH)])
            pltpu.sync_copy(ebuf, eacc.at[dbuf], add=True)
            if with_sim:
                pltpu.sync_copy(ssimb, ssim_hbm.at[pl.ds(base, CH)])
                pltpu.sync_copy(ssimb, sacc.at[sbuf], add=True)
            return 0

        lax.fori_loop(0, nch, chunk, 0)
        plsc.subcore_barrier()

        @pl.when(sid == 0)
        def _():
            pltpu.sync_copy(eacc, esum_hbm.at[cid])
            if with_sim:
                pltpu.sync_copy(sacc, ssum_hbm.at[cid])

    kern = functools.partial(
        pl.kernel,
        mesh=plsc.VectorSubcoreMesh(**_MESH),
        out_type=out_type,
        scratch_types=scratch,
        compiler_params=pltpu.CompilerParams(needs_layout_passes=False),
    )(body)
    if with_sim:
        return kern(src, dst, uu, pars, common)
    return kern(src, dst, uu, pars)


def _aggregate(src, dst, e, ssim, esum_p, ssum_p, xw, pars):
    nch = EFP // (NW * CH)
    KB = 128

    @functools.partial(
        pl.kernel,
        mesh=plsc.VectorSubcoreMesh(**_MESH),
        out_type=jax.ShapeDtypeStruct((NP * H,), jnp.float32),
        compiler_params=pltpu.CompilerParams(needs_layout_passes=False),
        scratch_types=[
            pltpu.VMEM((NP,), jnp.float32),
            pltpu.VMEM((NP,), jnp.float32),
            pltpu.VMEM((NP,), jnp.float32),
            pltpu.VMEM((RPT * H,), jnp.float32),
            pltpu.VMEM((CAP,), jnp.int32),
            pltpu.VMEM((CAP,), jnp.float32),
            pltpu.VMEM((CAP,), jnp.int32),
            pltpu.VMEM((KB, H), jnp.float32),
            pltpu.VMEM((CH,), jnp.int32),
            pltpu.VMEM((CH,), jnp.int32),
            pltpu.VMEM((CH,), jnp.float32),
            pltpu.VMEM((CH,), jnp.float32),
            pltpu.VMEM((16,), jnp.float32),
            pltpu.SemaphoreType.DMA,
        ],
    )
    def k(src_hbm, dst_hbm, e_hbm, ssim_hbm, esum_hbm, ssum_hbm, xw_hbm,
          pars_hbm, agg_hbm,
          esumt, ssumt, tmpt, acc, csrc, ccoef, coff, rows,
          sbuf, dbuf, ebuf, ssimb, parv, sem):
        wid = _wid()
        lo = wid * RPT
        base0 = wid * nch * CH

        pltpu.sync_copy(pars_hbm, parv)
        al = jnp.minimum(jnp.maximum(parv[pl.ds(0, 16)][0], 0.0001), 0.9999)
        al1 = 1.0 - al

        pltpu.sync_copy(esum_hbm.at[0], esumt)
        pltpu.sync_copy(esum_hbm.at[1], tmpt)

        def addt(i, _):
            sl = pl.ds(i * 16, 16)
            esumt[sl] = esumt[sl] + tmpt[sl]
            return 0

        lax.fori_loop(0, NP // 16, addt, 0)
        pltpu.sync_copy(ssum_hbm.at[0], ssumt)
        pltpu.sync_copy(ssum_hbm.at[1], tmpt)

        def addt2(i, _):
            sl = pl.ds(i * 16, 16)
            ssumt[sl] = ssumt[sl] + tmpt[sl]
            return 0

        lax.fori_loop(0, NP // 16, addt2, 0)

        def initacc(i, _):
            acc[pl.ds(i * 16, 16)] = jnp.full((16,), NEG, jnp.float32)
            return 0

        lax.fori_loop(0, RPT * H // 16, initacc, 0)

        def initsrc(i, _):
            csrc[pl.ds(i * 16, 16)] = jnp.zeros((16,), jnp.int32)
            return 0

        lax.fori_loop(0, CAP // 16, initsrc, 0)

        def chunk(c, cnt):
            base = base0 + c * CH
            pltpu.sync_copy(src_hbm.at[pl.ds(base, CH)], sbuf)
            pltpu.sync_copy(dst_hbm.at[pl.ds(base, CH)], dbuf)
            pltpu.sync_copy(e_hbm.at[pl.ds(base, CH)], ebuf)
            pltpu.sync_copy(ssim_hbm.at[pl.ds(base, CH)], ssimb)
            for v in range(CH // 16):
                sl = pl.ds(v * 16, 16)
                s = sbuf[sl]
                d = dbuf[sl]
                m = (d >= lo) & (d < lo + RPT) & (d < N)
                es = plsc.load_gather(esumt, [d])
                ss = plsc.load_gather(ssumt, [d])
                coef = al * ebuf[sl] / es + al1 * ssimb[sl] / ss
                off = (d - lo) * H
                plsc.store_compressed(csrc.at[pl.ds(cnt, 16)], s, mask=m)
                plsc.store_compressed(ccoef.at[pl.ds(cnt, 16)], coef, mask=m)
                plsc.store_compressed(coff.at[pl.ds(cnt, 16)], off, mask=m)
                cnt = jnp.minimum(
                    cnt + jnp.sum(m.astype(jnp.int32)), CAP - 16)
            return cnt

        cnt = lax.fori_loop(0, nch, chunk, jnp.int32(0))

        def batch(b, _):
            pltpu.async_copy(
                xw_hbm.at[csrc.at[pl.ds(b * KB, KB)]], rows, sem).wait()
            nb = jnp.minimum(cnt - b * KB, KB)

            def slot(t, _):
                off = coff[pl.ds(b * KB + t, 16)][0]
                cf = ccoef[pl.ds(b * KB + t, 16)][0]
                for j in range(H // 16):
                    sl = pl.ds(off + j * 16, 16)
                    acc[sl] = jnp.maximum(acc[sl],
                                          rows[t, pl.ds(j * 16, 16)] * cf)
                return 0

            lax.fori_loop(0, nb, slot, 0)
            return 0

        lax.fori_loop(0, (cnt + KB - 1) // KB, batch, 0)
        pltpu.sync_copy(acc, agg_hbm.at[pl.ds(lo * H, RPT * H)])

    return k(src, dst, e, ssim, esum_p, ssum_p, xw, pars)


def kernel(x, edge_index, W1, Wa1, ba1, alpha1, b1, W2, Wa2, ba2, alpha2, b2,
           Wout, bout):
    loops = jnp.arange(N, dtype=edge_index.dtype)
    src = jnp.concatenate([edge_index[0], loops])
    dst = jnp.concatenate([edge_index[1], loops])
    npad = EFP - src.shape[0]
    fill = jnp.full((npad,), N, edge_index.dtype)
    src = jnp.concatenate([src, fill])
    dst = jnp.concatenate([dst, fill])

    xp = jnp.pad(x, ((0, NP - N), (0, 0)))

    a_ref = jax.new_ref(jnp.zeros((NP * NP,), jnp.float32))
    _scatter_adj(src, dst, a_ref)
    A = a_ref[...].reshape(NP, NP)
    common = _common_matmul(_to_bf16(A)).reshape(-1)

    def pars16(*vals):
        v = jnp.concatenate([jnp.asarray(t, jnp.float32).reshape(-1)
                             for t in vals])
        return jnp.pad(v, (0, 16 - v.shape[0]))

    xw1, uu1 = _node_transform(xp, W1, Wa1)
    e1, esum1, ssim, ssum = _edge_scalars(src, dst, common, uu1,
                                          pars16(ba1), with_sim=True)
    agg1 = _aggregate(src, dst, e1, ssim, esum1, ssum, xw1, pars16(alpha1))

    xw2, uu2 = _node_transform(agg1.reshape(NP, H), W2, Wa2,
                               bias=b1, relu=True)
    e2, esum2 = _edge_scalars(src, dst, None, uu2, pars16(ba2),
                              with_sim=False)
    agg2 = _aggregate(src, dst, e2, ssim, esum2, ssum, xw2, pars16(alpha2))

    out = _final(agg2.reshape(NP, H), b2, Wout, bout)
    return out[:N]
